# baseline (speedup 1.0000x reference)
import functools

import jax
import jax.numpy as jnp
from jax import lax
from jax.experimental import pallas as pl
from jax.experimental.pallas import tpu as pltpu
from jax.experimental.pallas import tpu_sc as plsc

N = 10000
E = 160000
ATOM_DIM = 128
BOND_DIM = 16
H = 512
L = 4
G = 16
BN_EPS = 1e-5

HC = 128
NCH = H // HC
NBLK = 400
NNB = N // NBLK
NSUB = 16
EB = 64
BPT = 80
E_PAD = 2 * NSUB * BPT * EB
EPC = E_PAD // 2
EPTC = BPT * EB
NSPM = N + 48
ZSTR = NSPM // EB
OSTR = N // 80
EBLK = 2048
NEB = E_PAD // EBLK


def _fold_body(wee_ref, wlin_ref, bee_ref, blin_ref, out_ref):
    wl = wlin_ref[0]
    wc = jnp.dot(wee_ref[...], wl, preferred_element_type=jnp.float32, precision=lax.Precision.HIGHEST)
    bc = jnp.dot(bee_ref[...], wl, preferred_element_type=jnp.float32, precision=lax.Precision.HIGHEST) + blin_ref[0]
    out_ref[0] = jnp.concatenate(
        [wc, bc, jnp.zeros((7, HC), jnp.float32)], axis=0)


def _h0_body(x_ref, w_ref, b_ref, out_ref):
    out_ref[0] = (
        jnp.dot(x_ref[...], w_ref[...], preferred_element_type=jnp.float32, precision=lax.Precision.HIGHEST)
        + b_ref[...])


def _el_body(ea_ref, w_ref, out_ref):
    w = w_ref[0]
    out_ref[0] = (
        jnp.dot(ea_ref[...], w[:BOND_DIM], preferred_element_type=jnp.float32, precision=lax.Precision.HIGHEST)
        + w[BOND_DIM:BOND_DIM + 1])


def _mlp1_body(h_ref, a_ref, a2_ref, w1_ref, b1_ref, w2_ref, b2_ref,
               z_ref, sums_ref):
    i = pl.program_id(0)
    t = (jnp.concatenate([h_ref[k] for k in range(NCH)], axis=1)
         + jnp.concatenate([a_ref[k] for k in range(NCH)], axis=1)
         + jnp.concatenate([a2_ref[k] for k in range(NCH)], axis=1))
    a1 = jnp.maximum(
        jnp.dot(t, w1_ref[0], preferred_element_type=jnp.float32, precision=lax.Precision.HIGHEST) + b1_ref[0],
        0.0)
    z = jnp.dot(a1, w2_ref[0], preferred_element_type=jnp.float32, precision=lax.Precision.HIGHEST) + b2_ref[0]
    z_ref[...] = z
    part = jnp.concatenate(
        [jnp.sum(z, axis=0, keepdims=True),
         jnp.sum(z * z, axis=0, keepdims=True),
         jnp.zeros((6, H), jnp.float32)], axis=0)

    @pl.when(i == 0)
    def _():
        sums_ref[...] = part

    @pl.when(i > 0)
    def _():
        sums_ref[...] = sums_ref[...] + part


def _mlp2_body(z_ref, sums_ref, h_ref, g_ref, be_ref, out_ref):
    srow = sums_ref[...]
    mu = srow[0:1] * (1.0 / N)
    var = srow[1:2] * (1.0 / N) - mu * mu
    inv = jax.lax.rsqrt(var + BN_EPS)
    zn = (z_ref[...] - mu) * inv * g_ref[0] + be_ref[0]
    out_ref[0] = h_ref[0] + jnp.maximum(zn, 0.0)


def _pool_body(h_ref, b_ref, hout_ref, repr_ref, acc_ref, cnt_ref):
    i = pl.program_id(0)
    hb = jnp.concatenate([h_ref[k] for k in range(NCH)], axis=1)
    hout_ref[...] = hb
    bv = b_ref[0]
    oh = (lax.broadcasted_iota(jnp.int32, (G, NBLK), 0) == bv).astype(jnp.float32)
    part = jnp.dot(oh, hb, preferred_element_type=jnp.float32, precision=lax.Precision.HIGHEST)
    pcnt = jnp.sum(oh, axis=1, keepdims=True)

    @pl.when(i == 0)
    def _():
        acc_ref[...] = part
        cnt_ref[...] = jnp.broadcast_to(pcnt, (G, HC))

    @pl.when(i > 0)
    def _():
        acc_ref[...] = acc_ref[...] + part
        cnt_ref[...] = cnt_ref[...] + jnp.broadcast_to(pcnt, (G, HC))

    repr_ref[...] = acc_ref[...] / jnp.clip(cnt_ref[:, 0:1], 1.0, None)


def _make_sc_edge(l):
    mesh = plsc.VectorSubcoreMesh(core_axis_name="c", subcore_axis_name="s")

    @functools.partial(
        pl.kernel,
        out_type=[jax.ShapeDtypeStruct((NCH, N, HC), jnp.float32),
                  jax.ShapeDtypeStruct((NCH, N, HC), jnp.float32)],
        mesh=mesh,
        scratch_types=[
            pltpu.VMEM((BPT, EB), jnp.int32),
            pltpu.VMEM((BPT, EB), jnp.int32),
            pltpu.VMEM((EB, HC), jnp.float32),
            pltpu.VMEM((EB, HC), jnp.float32),
            pltpu.VMEM((EB, HC), jnp.float32),
            pltpu.VMEM_SHARED((NSPM, HC), jnp.float32),
            pltpu.SemaphoreType.DMA,
            pltpu.SemaphoreType.DMA,
        ],
    )
    def sc_edge(h_hbm, el_hbm, src_hbm, dst_hbm, out0_hbm, out1_hbm,
                sidx, didx, gbuf, ebuf, zbuf, spm, gsem, esem):
        c = lax.axis_index("c")
        s = lax.axis_index("s")
        pltpu.sync_copy(src_hbm.at[c].at[s], sidx)
        pltpu.sync_copy(dst_hbm.at[c].at[s], didx)

        @pl.loop(0, EB)
        def _(r):
            for j in range(0, HC, 16):
                zbuf[r, pl.ds(j, 16)] = jnp.zeros((16,), jnp.float32)

        for q in range(NCH):
            lq = l * NCH + q
            cbase = c * EPC + s * EPTC
            for t in range(10):
                sid = s + t * NSUB

                @pl.when(sid < ZSTR)
                def _():
                    pltpu.sync_copy(zbuf, spm.at[pl.ds(sid * EB, EB)])
            plsc.subcore_barrier()

            pltpu.async_copy(h_hbm.at[q].at[sidx.at[0]], gbuf, gsem)

            @pl.loop(0, BPT)
            def _(b):
                cp2 = pltpu.async_copy(
                    el_hbm.at[lq].at[pl.ds(cbase + b * EB, EB)], ebuf, esem)
                pltpu.make_async_copy(h_hbm.at[q].at[sidx.at[0]],
                                      gbuf, gsem).wait()
                cp2.wait()

                @pl.loop(0, EB)
                def _(r):
                    for j in range(0, HC, 16):
                        g = gbuf[r, pl.ds(j, 16)]
                        e = ebuf[r, pl.ds(j, 16)]
                        ebuf[r, pl.ds(j, 16)] = jnp.maximum(g + e, 0.0)

                bi = jnp.minimum(b + 1, BPT - 1)
                pltpu.async_copy(h_hbm.at[q].at[sidx.at[bi]], gbuf, gsem)
                pltpu.sync_copy(ebuf, spm.at[didx.at[b]], add=True)

            pltpu.make_async_copy(h_hbm.at[q].at[sidx.at[0]],
                                  gbuf, gsem).wait()
            plsc.subcore_barrier()
            for t in range(8):
                sid = s + t * NSUB

                @pl.when((sid < OSTR) & (c == 0))
                def _():
                    pltpu.sync_copy(spm.at[pl.ds(sid * 80, 80)],
                                    out0_hbm.at[q].at[pl.ds(sid * 80, 80)])

                @pl.when((sid < OSTR) & (c == 1))
                def _():
                    pltpu.sync_copy(spm.at[pl.ds(sid * 80, 80)],
                                    out1_hbm.at[q].at[pl.ds(sid * 80, 80)])

    return sc_edge


_SC_EDGE_CACHE = {}


def _sc_edge(l):
    if l not in _SC_EDGE_CACHE:
        _SC_EDGE_CACHE[l] = _make_sc_edge(l)
    return _SC_EDGE_CACHE[l]


def kernel(x, edge_index, edge_attr, batch, W_atom, b_atom, W_ee, b_ee,
           W_lin, b_lin, W1, b1, W2, b2, gamma, beta):
    f32 = jnp.float32
    src4d = jnp.pad(edge_index[0].astype(jnp.int32),
                    (0, E_PAD - E)).reshape(2, NSUB, BPT, EB)
    dst4d = jnp.pad(edge_index[1].astype(jnp.int32), (0, E_PAD - E),
                    constant_values=N).reshape(2, NSUB, BPT, EB)
    ea_pad = jnp.pad(edge_attr, ((0, E_PAD - E), (0, 0)))
    b_atom2 = b_atom.reshape(1, H)
    b_ee2 = b_ee.reshape(1, H)
    b_lin2 = b_lin.reshape(L, 1, H)
    b1_2 = b1.reshape(L, 1, 2 * H)
    b2_2 = b2.reshape(L, 1, H)
    gamma2 = gamma.reshape(L, 1, H)
    beta2 = beta.reshape(L, 1, H)
    batch3 = batch.astype(jnp.int32).reshape(NNB, 1, NBLK)

    wc_aug = pl.pallas_call(
        _fold_body,
        grid=(L, NCH),
        in_specs=[
            pl.BlockSpec((BOND_DIM, H), lambda l, c: (0, 0)),
            pl.BlockSpec((1, H, HC), lambda l, c: (l, 0, c)),
            pl.BlockSpec((1, H), lambda l, c: (0, 0)),
            pl.BlockSpec((1, 1, HC), lambda l, c: (l, 0, c)),
        ],
        out_specs=pl.BlockSpec((1, 24, HC), lambda l, c: (l * NCH + c, 0, 0)),
        out_shape=jax.ShapeDtypeStruct((L * NCH, 24, HC), f32),
    )(W_ee, W_lin, b_ee2, b_lin2)

    h = pl.pallas_call(
        _h0_body,
        grid=(NNB, NCH),
        in_specs=[
            pl.BlockSpec((NBLK, ATOM_DIM), lambda i, c: (i, 0)),
            pl.BlockSpec((ATOM_DIM, HC), lambda i, c: (0, c)),
            pl.BlockSpec((1, HC), lambda i, c: (0, c)),
        ],
        out_specs=pl.BlockSpec((1, NBLK, HC), lambda i, c: (c, i, 0)),
        out_shape=jax.ShapeDtypeStruct((NCH, N, HC), f32),
    )(x, W_atom, b_atom2)

    el = pl.pallas_call(
        _el_body,
        grid=(L * NCH, NEB),
        in_specs=[
            pl.BlockSpec((EBLK, BOND_DIM), lambda lc, e: (e, 0)),
            pl.BlockSpec((1, 24, HC), lambda lc, e: (lc, 0, 0)),
        ],
        out_specs=pl.BlockSpec((1, EBLK, HC), lambda lc, e: (lc, e, 0)),
        out_shape=jax.ShapeDtypeStruct((L * NCH, E_PAD, HC), f32),
    )(ea_pad, wc_aug)

    for l in range(L):
        aggr0, aggr1 = _sc_edge(l)(h, el, src4d, dst4d)

        z, sums = pl.pallas_call(
            _mlp1_body,
            grid=(NNB,),
            in_specs=[
                pl.BlockSpec((NCH, NBLK, HC), lambda i: (0, i, 0)),
                pl.BlockSpec((NCH, NBLK, HC), lambda i: (0, i, 0)),
                pl.BlockSpec((NCH, NBLK, HC), lambda i: (0, i, 0)),
                pl.BlockSpec((1, H, 2 * H), lambda i, l=l: (l, 0, 0)),
                pl.BlockSpec((1, 1, 2 * H), lambda i, l=l: (l, 0, 0)),
                pl.BlockSpec((1, 2 * H, H), lambda i, l=l: (l, 0, 0)),
                pl.BlockSpec((1, 1, H), lambda i, l=l: (l, 0, 0)),
            ],
            out_specs=[
                pl.BlockSpec((NBLK, H), lambda i: (i, 0)),
                pl.BlockSpec((8, H), lambda i: (0, 0)),
            ],
            out_shape=[
                jax.ShapeDtypeStruct((N, H), f32),
                jax.ShapeDtypeStruct((8, H), f32),
            ],
        )(h, aggr0, aggr1, W1, b1_2, W2, b2_2)

        h = pl.pallas_call(
            _mlp2_body,
            grid=(NNB, NCH),
            in_specs=[
                pl.BlockSpec((NBLK, HC), lambda i, c: (i, c)),
                pl.BlockSpec((8, HC), lambda i, c: (0, c)),
                pl.BlockSpec((1, NBLK, HC), lambda i, c: (c, i, 0)),
                pl.BlockSpec((1, 1, HC), lambda i, c, l=l: (l, 0, c)),
                pl.BlockSpec((1, 1, HC), lambda i, c, l=l: (l, 0, c)),
            ],
            out_specs=pl.BlockSpec((1, NBLK, HC), lambda i, c: (c, i, 0)),
            out_shape=jax.ShapeDtypeStruct((NCH, N, HC), f32),
        )(z, sums, h, gamma2, beta2)

    h_out, graph_repr = pl.pallas_call(
        _pool_body,
        grid=(NNB,),
        in_specs=[
            pl.BlockSpec((NCH, NBLK, HC), lambda i: (0, i, 0)),
            pl.BlockSpec((1, 1, NBLK), lambda i: (i, 0, 0)),
        ],
        out_specs=[
            pl.BlockSpec((NBLK, H), lambda i: (i, 0)),
            pl.BlockSpec((G, H), lambda i: (0, 0)),
        ],
        out_shape=[
            jax.ShapeDtypeStruct((N, H), f32),
            jax.ShapeDtypeStruct((G, H), f32),
        ],
        scratch_shapes=[
            pltpu.VMEM((G, H), f32),
            pltpu.VMEM((G, HC), f32),
        ],
    )(h, batch3)

    return (h_out, graph_repr)


# speedup vs baseline: 1.1066x; 1.1066x over previous
import functools

import jax
import jax.numpy as jnp
from jax import lax
from jax.experimental import pallas as pl
from jax.experimental.pallas import tpu as pltpu
from jax.experimental.pallas import tpu_sc as plsc

N = 10000
E = 160000
ATOM_DIM = 128
BOND_DIM = 16
H = 512
L = 4
G = 16
BN_EPS = 1e-5

HC = 128
NCH = H // HC
NBLK = 400
NNB = N // NBLK
NSUB = 16
EB = 64
BPT = 80
E_PAD = 2 * NSUB * BPT * EB
EPC = E_PAD // 2
EPTC = BPT * EB
NSPM = N + 48
ZSTR = NSPM // EB
OSTR = N // 80
EBLK = 2048
NEB = E_PAD // EBLK


def _fold_body(wee_ref, wlin_ref, bee_ref, blin_ref, out_ref):
    wl = wlin_ref[0]
    wc = jnp.dot(wee_ref[...], wl, preferred_element_type=jnp.float32)
    bc = jnp.dot(bee_ref[...], wl, preferred_element_type=jnp.float32) + blin_ref[0]
    out_ref[0] = jnp.concatenate(
        [wc, bc, jnp.zeros((7, HC), jnp.float32)], axis=0)


def _h0_body(x_ref, w_ref, b_ref, out_ref):
    out_ref[0] = (
        jnp.dot(x_ref[...], w_ref[...], preferred_element_type=jnp.float32)
        + b_ref[...])


def _el_body(ea_ref, w_ref, out_ref):
    w = w_ref[0]
    out_ref[0] = (
        jnp.dot(ea_ref[...], w[:BOND_DIM], preferred_element_type=jnp.float32)
        + w[BOND_DIM:BOND_DIM + 1])


def _mlp1_body(h_ref, a_ref, a2_ref, w1_ref, b1_ref, w2_ref, b2_ref,
               z_ref, sums_ref):
    i = pl.program_id(0)
    t = (jnp.concatenate([h_ref[k] for k in range(NCH)], axis=1)
         + jnp.concatenate([a_ref[k] for k in range(NCH)], axis=1)
         + jnp.concatenate([a2_ref[k] for k in range(NCH)], axis=1))
    a1 = jnp.maximum(
        jnp.dot(t, w1_ref[0], preferred_element_type=jnp.float32) + b1_ref[0],
        0.0)
    z = jnp.dot(a1, w2_ref[0], preferred_element_type=jnp.float32) + b2_ref[0]
    z_ref[...] = z
    part = jnp.concatenate(
        [jnp.sum(z, axis=0, keepdims=True),
         jnp.sum(z * z, axis=0, keepdims=True),
         jnp.zeros((6, H), jnp.float32)], axis=0)

    @pl.when(i == 0)
    def _():
        sums_ref[...] = part

    @pl.when(i > 0)
    def _():
        sums_ref[...] = sums_ref[...] + part


def _mlp2_body(z_ref, sums_ref, h_ref, g_ref, be_ref, out_ref):
    srow = sums_ref[...]
    mu = srow[0:1] * (1.0 / N)
    var = srow[1:2] * (1.0 / N) - mu * mu
    inv = jax.lax.rsqrt(var + BN_EPS)
    zn = (z_ref[...] - mu) * inv * g_ref[0] + be_ref[0]
    out_ref[0] = h_ref[0] + jnp.maximum(zn, 0.0)


def _pool_body(h_ref, b_ref, hout_ref, repr_ref, acc_ref, cnt_ref):
    i = pl.program_id(0)
    hb = jnp.concatenate([h_ref[k] for k in range(NCH)], axis=1)
    hout_ref[...] = hb
    bv = b_ref[0]
    oh = (lax.broadcasted_iota(jnp.int32, (G, NBLK), 0) == bv).astype(jnp.float32)
    part = jnp.dot(oh, hb, preferred_element_type=jnp.float32)
    pcnt = jnp.sum(oh, axis=1, keepdims=True)

    @pl.when(i == 0)
    def _():
        acc_ref[...] = part
        cnt_ref[...] = jnp.broadcast_to(pcnt, (G, HC))

    @pl.when(i > 0)
    def _():
        acc_ref[...] = acc_ref[...] + part
        cnt_ref[...] = cnt_ref[...] + jnp.broadcast_to(pcnt, (G, HC))

    repr_ref[...] = acc_ref[...] / jnp.clip(cnt_ref[:, 0:1], 1.0, None)


def _make_sc_edge(l):
    mesh = plsc.VectorSubcoreMesh(core_axis_name="c", subcore_axis_name="s")

    @functools.partial(
        pl.kernel,
        out_type=[jax.ShapeDtypeStruct((NCH, N, HC), jnp.float32),
                  jax.ShapeDtypeStruct((NCH, N, HC), jnp.float32)],
        mesh=mesh,
        scratch_types=[
            pltpu.VMEM((BPT, EB), jnp.int32),
            pltpu.VMEM((BPT, EB), jnp.int32),
            pltpu.VMEM((EB, HC), jnp.float32),
            pltpu.VMEM((EB, HC), jnp.float32),
            pltpu.VMEM((EB, HC), jnp.float32),
            pltpu.VMEM_SHARED((NSPM, HC), jnp.float32),
            pltpu.SemaphoreType.DMA,
            pltpu.SemaphoreType.DMA,
        ],
    )
    def sc_edge(h_hbm, el_hbm, src_hbm, dst_hbm, out0_hbm, out1_hbm,
                sidx, didx, gbuf, ebuf, zbuf, spm, gsem, esem):
        c = lax.axis_index("c")
        s = lax.axis_index("s")
        pltpu.sync_copy(src_hbm.at[c].at[s], sidx)
        pltpu.sync_copy(dst_hbm.at[c].at[s], didx)

        @pl.loop(0, EB)
        def _(r):
            for j in range(0, HC, 16):
                zbuf[r, pl.ds(j, 16)] = jnp.zeros((16,), jnp.float32)

        for q in range(NCH):
            lq = l * NCH + q
            cbase = c * EPC + s * EPTC
            for t in range(10):
                sid = s + t * NSUB

                @pl.when(sid < ZSTR)
                def _():
                    pltpu.sync_copy(zbuf, spm.at[pl.ds(sid * EB, EB)])
            plsc.subcore_barrier()

            pltpu.async_copy(h_hbm.at[q].at[sidx.at[0]], gbuf, gsem)

            @pl.loop(0, BPT)
            def _(b):
                cp2 = pltpu.async_copy(
                    el_hbm.at[lq].at[pl.ds(cbase + b * EB, EB)], ebuf, esem)
                pltpu.make_async_copy(h_hbm.at[q].at[sidx.at[0]],
                                      gbuf, gsem).wait()
                cp2.wait()

                @pl.loop(0, EB)
                def _(r):
                    for j in range(0, HC, 16):
                        g = gbuf[r, pl.ds(j, 16)]
                        e = ebuf[r, pl.ds(j, 16)]
                        ebuf[r, pl.ds(j, 16)] = jnp.maximum(g + e, 0.0)

                bi = jnp.minimum(b + 1, BPT - 1)
                pltpu.async_copy(h_hbm.at[q].at[sidx.at[bi]], gbuf, gsem)
                pltpu.sync_copy(ebuf, spm.at[didx.at[b]], add=True)

            pltpu.make_async_copy(h_hbm.at[q].at[sidx.at[0]],
                                  gbuf, gsem).wait()
            plsc.subcore_barrier()
            for t in range(8):
                sid = s + t * NSUB

                @pl.when((sid < OSTR) & (c == 0))
                def _():
                    pltpu.sync_copy(spm.at[pl.ds(sid * 80, 80)],
                                    out0_hbm.at[q].at[pl.ds(sid * 80, 80)])

                @pl.when((sid < OSTR) & (c == 1))
                def _():
                    pltpu.sync_copy(spm.at[pl.ds(sid * 80, 80)],
                                    out1_hbm.at[q].at[pl.ds(sid * 80, 80)])

            plsc.subcore_barrier()

    return sc_edge


_SC_EDGE_CACHE = {}


def _sc_edge(l):
    if l not in _SC_EDGE_CACHE:
        _SC_EDGE_CACHE[l] = _make_sc_edge(l)
    return _SC_EDGE_CACHE[l]


def kernel(x, edge_index, edge_attr, batch, W_atom, b_atom, W_ee, b_ee,
           W_lin, b_lin, W1, b1, W2, b2, gamma, beta):
    f32 = jnp.float32
    src4d = jnp.pad(edge_index[0].astype(jnp.int32),
                    (0, E_PAD - E)).reshape(2, NSUB, BPT, EB)
    dst4d = jnp.pad(edge_index[1].astype(jnp.int32), (0, E_PAD - E),
                    constant_values=N).reshape(2, NSUB, BPT, EB)
    ea_pad = jnp.pad(edge_attr, ((0, E_PAD - E), (0, 0)))
    b_atom2 = b_atom.reshape(1, H)
    b_ee2 = b_ee.reshape(1, H)
    b_lin2 = b_lin.reshape(L, 1, H)
    b1_2 = b1.reshape(L, 1, 2 * H)
    b2_2 = b2.reshape(L, 1, H)
    gamma2 = gamma.reshape(L, 1, H)
    beta2 = beta.reshape(L, 1, H)
    batch3 = batch.astype(jnp.int32).reshape(NNB, 1, NBLK)

    wc_aug = pl.pallas_call(
        _fold_body,
        grid=(L, NCH),
        in_specs=[
            pl.BlockSpec((BOND_DIM, H), lambda l, c: (0, 0)),
            pl.BlockSpec((1, H, HC), lambda l, c: (l, 0, c)),
            pl.BlockSpec((1, H), lambda l, c: (0, 0)),
            pl.BlockSpec((1, 1, HC), lambda l, c: (l, 0, c)),
        ],
        out_specs=pl.BlockSpec((1, 24, HC), lambda l, c: (l * NCH + c, 0, 0)),
        out_shape=jax.ShapeDtypeStruct((L * NCH, 24, HC), f32),
    )(W_ee, W_lin, b_ee2, b_lin2)

    h = pl.pallas_call(
        _h0_body,
        grid=(NNB, NCH),
        in_specs=[
            pl.BlockSpec((NBLK, ATOM_DIM), lambda i, c: (i, 0)),
            pl.BlockSpec((ATOM_DIM, HC), lambda i, c: (0, c)),
            pl.BlockSpec((1, HC), lambda i, c: (0, c)),
        ],
        out_specs=pl.BlockSpec((1, NBLK, HC), lambda i, c: (c, i, 0)),
        out_shape=jax.ShapeDtypeStruct((NCH, N, HC), f32),
    )(x, W_atom, b_atom2)

    el = pl.pallas_call(
        _el_body,
        grid=(L * NCH, NEB),
        in_specs=[
            pl.BlockSpec((EBLK, BOND_DIM), lambda lc, e: (e, 0)),
            pl.BlockSpec((1, 24, HC), lambda lc, e: (lc, 0, 0)),
        ],
        out_specs=pl.BlockSpec((1, EBLK, HC), lambda lc, e: (lc, e, 0)),
        out_shape=jax.ShapeDtypeStruct((L * NCH, E_PAD, HC), f32),
    )(ea_pad, wc_aug)

    for l in range(L):
        aggr0, aggr1 = _sc_edge(l)(h, el, src4d, dst4d)

        z, sums = pl.pallas_call(
            _mlp1_body,
            grid=(NNB,),
            in_specs=[
                pl.BlockSpec((NCH, NBLK, HC), lambda i: (0, i, 0)),
                pl.BlockSpec((NCH, NBLK, HC), lambda i: (0, i, 0)),
                pl.BlockSpec((NCH, NBLK, HC), lambda i: (0, i, 0)),
                pl.BlockSpec((1, H, 2 * H), lambda i, l=l: (l, 0, 0)),
                pl.BlockSpec((1, 1, 2 * H), lambda i, l=l: (l, 0, 0)),
                pl.BlockSpec((1, 2 * H, H), lambda i, l=l: (l, 0, 0)),
                pl.BlockSpec((1, 1, H), lambda i, l=l: (l, 0, 0)),
            ],
            out_specs=[
                pl.BlockSpec((NBLK, H), lambda i: (i, 0)),
                pl.BlockSpec((8, H), lambda i: (0, 0)),
            ],
            out_shape=[
                jax.ShapeDtypeStruct((N, H), f32),
                jax.ShapeDtypeStruct((8, H), f32),
            ],
        )(h, aggr0, aggr1, W1, b1_2, W2, b2_2)

        h = pl.pallas_call(
            _mlp2_body,
            grid=(NNB, NCH),
            in_specs=[
                pl.BlockSpec((NBLK, HC), lambda i, c: (i, c)),
                pl.BlockSpec((8, HC), lambda i, c: (0, c)),
                pl.BlockSpec((1, NBLK, HC), lambda i, c: (c, i, 0)),
                pl.BlockSpec((1, 1, HC), lambda i, c, l=l: (l, 0, c)),
                pl.BlockSpec((1, 1, HC), lambda i, c, l=l: (l, 0, c)),
            ],
            out_specs=pl.BlockSpec((1, NBLK, HC), lambda i, c: (c, i, 0)),
            out_shape=jax.ShapeDtypeStruct((NCH, N, HC), f32),
        )(z, sums, h, gamma2, beta2)

    h_out, graph_repr = pl.pallas_call(
        _pool_body,
        grid=(NNB,),
        in_specs=[
            pl.BlockSpec((NCH, NBLK, HC), lambda i: (0, i, 0)),
            pl.BlockSpec((1, 1, NBLK), lambda i: (i, 0, 0)),
        ],
        out_specs=[
            pl.BlockSpec((NBLK, H), lambda i: (i, 0)),
            pl.BlockSpec((G, H), lambda i: (0, 0)),
        ],
        out_shape=[
            jax.ShapeDtypeStruct((N, H), f32),
            jax.ShapeDtypeStruct((G, H), f32),
        ],
        scratch_shapes=[
            pltpu.VMEM((G, H), f32),
            pltpu.VMEM((G, HC), f32),
        ],
    )(h, batch3)

    return (h_out, graph_repr)
